# baseline (speedup 1.0000x reference)
import jax
import jax.numpy as jnp
from jax import lax
from jax.experimental import pallas as pl
from jax.experimental.pallas import tpu as pltpu
from jax.experimental.pallas import tpu_sc as plsc

MEM = 20000
FEAT = 3 * 32 * 32
NCLS = 100
PK = 128
BATCH = 4096
COPY_ROWS = 512
KCHUNK = 512
NSUB = 32
WROWS = BATCH // NSUB
XSUB = 32

CPCH = 1000
PKCH = 2000

_vector_mesh = plsc.VectorSubcoreMesh(
    core_axis_name="core", subcore_axis_name="subcore")


def _make_tc_mesh():
    base = pltpu.create_tensorcore_mesh("tc")

    class _HbmTCMesh(type(base)):
        @property
        def default_memory_space(self):
            return pltpu.HBM

    return _HbmTCMesh(devices=base.devices, axis_names=base.axis_names)


def _kmap_body(idx_col_ref, idx_row_ref, out_ref):
    own = idx_col_ref[...]
    allv = idx_row_ref[...]
    iota = lax.broadcasted_iota(jnp.int32, (KCHUNK, BATCH), 1)
    sel = jnp.where(own == allv, iota, -1)
    out_ref[...] = jnp.max(sel, axis=1, keepdims=True)


def _pipe_copy(src, dst, bufs, sems, nch, rows):
    def rd(k):
        sl = pl.ds(k * rows, rows)
        return pltpu.make_async_copy(src.at[sl], bufs[k % 2], sems[k % 2])

    def wr(k):
        sl = pl.ds(k * rows, rows)
        return pltpu.make_async_copy(bufs[k % 2], dst.at[sl], sems[k % 2])

    rd(0).start()
    for k in range(nch):
        rd(k).wait()
        wr(k).start()
        if k + 1 < nch:
            if k >= 1:
                wr(k - 1).wait()
            rd(k + 1).start()
    if nch >= 2:
        wr(nch - 2).wait()
    wr(nch - 1).wait()


def _copy_dma_body(bx_hbm, pkb_hbm, obx_hbm, opk_hbm,
                   xa, xb, pa, pb, sem_a, sem_b, sem_c, sem_d):
    _pipe_copy(bx_hbm, obx_hbm, (xa, xb), (sem_a, sem_b),
               MEM // CPCH, CPCH)
    _pipe_copy(pkb_hbm, opk_hbm, (pa, pb), (sem_c, sem_d),
               MEM // PKCH, PKCH)


def _make_copy():
    return pl.pallas_call(
        _copy_dma_body,
        in_specs=[
            pl.BlockSpec(memory_space=pl.ANY),
            pl.BlockSpec(memory_space=pl.ANY),
        ],
        out_specs=[
            pl.BlockSpec(memory_space=pl.ANY),
            pl.BlockSpec(memory_space=pl.ANY),
        ],
        out_shape=[
            jax.ShapeDtypeStruct((MEM, FEAT), jnp.float32),
            jax.ShapeDtypeStruct((MEM, PK), jnp.int32),
        ],
        scratch_shapes=[
            pltpu.VMEM((CPCH, FEAT), jnp.float32),
            pltpu.VMEM((CPCH, FEAT), jnp.float32),
            pltpu.VMEM((PKCH, PK), jnp.int32),
            pltpu.VMEM((PKCH, PK), jnp.int32),
            pltpu.SemaphoreType.DMA,
            pltpu.SemaphoreType.DMA,
            pltpu.SemaphoreType.DMA,
            pltpu.SemaphoreType.DMA,
        ],
    )


def _make_sc_scatter():
    def body(idx_hbm, kmap_hbm, x_hbm, pkin_hbm, obx_ref, opk_ref,
             iw_vmem, kw_vmem, xw_vmem, pkw_vmem):
        core = lax.axis_index("core")
        sub = lax.axis_index("subcore")
        off = (core * 16 + sub) * WROWS

        pltpu.sync_copy(idx_hbm.at[0, pl.ds(off, WROWS)], iw_vmem)
        pltpu.sync_copy(kmap_hbm.at[0, pl.ds(off, WROWS)], kw_vmem)

        pltpu.sync_copy(pkin_hbm.at[kw_vmem], pkw_vmem)
        pltpu.sync_copy(pkw_vmem, opk_ref.at[iw_vmem])

        for k in range(WROWS // XSUB):
            sl = pl.ds(k * XSUB, XSUB)
            pltpu.sync_copy(x_hbm.at[kw_vmem.at[sl]], xw_vmem)
            pltpu.sync_copy(xw_vmem, obx_ref.at[iw_vmem.at[sl]])

    return pl.kernel(
        body,
        out_type=(),
        mesh=_vector_mesh,
        scratch_types=[
            pltpu.VMEM((WROWS,), jnp.int32),
            pltpu.VMEM((WROWS,), jnp.int32),
            pltpu.VMEM((XSUB, FEAT), jnp.float32),
            pltpu.VMEM((WROWS, PK), jnp.int32),
        ],
    )


def kernel(x, y, logits, t, idx, bx, by, bt, blogits):
    xf = x.reshape(BATCH, FEAT)
    bxf = bx.reshape(MEM, FEAT)

    logits_bits = jax.lax.bitcast_convert_type(logits, jnp.int32)
    t_col = jnp.full((BATCH, 1), t, dtype=jnp.int32)
    pad_in = jnp.zeros((BATCH, PK - NCLS - 2), jnp.int32)
    pk_in = jnp.concatenate([logits_bits, y[:, None], t_col, pad_in], axis=1)

    blogits_bits = jax.lax.bitcast_convert_type(blogits, jnp.int32)
    pad_buf = jnp.zeros((MEM, PK - NCLS - 2), jnp.int32)
    pk_buf = jnp.concatenate(
        [blogits_bits, by[:, None], bt[:, None], pad_buf], axis=1)

    kmap = pl.pallas_call(
        _kmap_body,
        grid=(BATCH // KCHUNK,),
        in_specs=[
            pl.BlockSpec((KCHUNK, 1), lambda i: (i, 0)),
            pl.BlockSpec((1, BATCH), lambda i: (0, 0)),
        ],
        out_specs=pl.BlockSpec((KCHUNK, 1), lambda i: (i, 0)),
        out_shape=jax.ShapeDtypeStruct((BATCH, 1), jnp.int32),
    )(idx[:, None], idx[None, :])

    cbx, cpk = _make_copy()(bxf, pk_buf)
    obx_ref = jax.new_ref(cbx)
    opk_ref = jax.new_ref(cpk)
    _make_sc_scatter()(idx[None, :], kmap.reshape(1, BATCH), xf, pk_in,
                       obx_ref, opk_ref)
    obx = obx_ref[...]
    opk = opk_ref[...]

    bx_new = obx.reshape(MEM, 3, 32, 32)
    blogits_new = jax.lax.bitcast_convert_type(opk[:, :NCLS], jnp.float32)
    by_new = opk[:, NCLS]
    bt_new = opk[:, NCLS + 1]
    return (bx_new, by_new, bt_new, blogits_new)


# speedup vs baseline: 1.0353x; 1.0353x over previous
import jax
import jax.numpy as jnp
from jax import lax
from jax.experimental import pallas as pl
from jax.experimental.pallas import tpu as pltpu
from jax.experimental.pallas import tpu_sc as plsc

MEM = 20000
FEAT = 3 * 32 * 32
NCLS = 100
PK = 128
BATCH = 4096
COPY_ROWS = 512
KCHUNK = 512
NSUB = 32
WROWS = BATCH // NSUB
XSUB = 32

HALF = MEM // 2
CPCH = 200
PKCH = 1000

_vector_mesh = plsc.VectorSubcoreMesh(
    core_axis_name="core", subcore_axis_name="subcore")


def _make_scalar_mesh():
    return plsc.ScalarSubcoreMesh(axis_name="score", num_cores=2)


def _kmap_body(idx_col_ref, idx_row_ref, out_ref):
    own = idx_col_ref[...]
    allv = idx_row_ref[...]
    iota = lax.broadcasted_iota(jnp.int32, (KCHUNK, BATCH), 1)
    sel = jnp.where(own == allv, iota, -1)
    out_ref[...] = jnp.max(sel, axis=1, keepdims=True)


def _pipe_copy(src, dst, bufs, sems, nch, rows, base):
    def rd(k):
        sl = pl.ds(base + k * rows, rows)
        return pltpu.make_async_copy(src.at[sl], bufs[k % 2], sems[k % 2])

    def wr(k):
        sl = pl.ds(base + k * rows, rows)
        return pltpu.make_async_copy(bufs[k % 2], dst.at[sl], sems[k % 2])

    rd(0).start()
    for k in range(nch):
        rd(k).wait()
        wr(k).start()
        if k + 1 < nch:
            if k >= 1:
                wr(k - 1).wait()
            rd(k + 1).start()
    if nch >= 2:
        wr(nch - 2).wait()
    wr(nch - 1).wait()


def _copy_dma_body(bx_hbm, pkb_hbm, obx_hbm, opk_hbm,
                   xa, xb, pa, pb, sem_a, sem_b, sem_c, sem_d):
    base = lax.axis_index("score") * HALF
    _pipe_copy(bx_hbm, obx_hbm, (xa, xb), (sem_a, sem_b),
               HALF // CPCH, CPCH, base)
    _pipe_copy(pkb_hbm, opk_hbm, (pa, pb), (sem_c, sem_d),
               HALF // PKCH, PKCH, base)


def _make_copy():
    return pl.kernel(
        _copy_dma_body,
        out_type=[
            jax.ShapeDtypeStruct((MEM, FEAT), jnp.float32),
            jax.ShapeDtypeStruct((MEM, PK), jnp.int32),
        ],
        mesh=_make_scalar_mesh(),
        scratch_types=[
            pltpu.VMEM_SHARED((CPCH, FEAT), jnp.float32),
            pltpu.VMEM_SHARED((CPCH, FEAT), jnp.float32),
            pltpu.VMEM_SHARED((PKCH, PK), jnp.int32),
            pltpu.VMEM_SHARED((PKCH, PK), jnp.int32),
            pltpu.SemaphoreType.DMA,
            pltpu.SemaphoreType.DMA,
            pltpu.SemaphoreType.DMA,
            pltpu.SemaphoreType.DMA,
        ],
    )


def _make_sc_scatter():
    def body(idx_hbm, kmap_hbm, x_hbm, pkin_hbm, obx_ref, opk_ref,
             iw_vmem, kw_vmem, xw_vmem, pkw_vmem):
        core = lax.axis_index("core")
        sub = lax.axis_index("subcore")
        off = (core * 16 + sub) * WROWS

        pltpu.sync_copy(idx_hbm.at[0, pl.ds(off, WROWS)], iw_vmem)
        pltpu.sync_copy(kmap_hbm.at[0, pl.ds(off, WROWS)], kw_vmem)

        pltpu.sync_copy(pkin_hbm.at[kw_vmem], pkw_vmem)
        pltpu.sync_copy(pkw_vmem, opk_ref.at[iw_vmem])

        for k in range(WROWS // XSUB):
            sl = pl.ds(k * XSUB, XSUB)
            pltpu.sync_copy(x_hbm.at[kw_vmem.at[sl]], xw_vmem)
            pltpu.sync_copy(xw_vmem, obx_ref.at[iw_vmem.at[sl]])

    return pl.kernel(
        body,
        out_type=(),
        mesh=_vector_mesh,
        scratch_types=[
            pltpu.VMEM((WROWS,), jnp.int32),
            pltpu.VMEM((WROWS,), jnp.int32),
            pltpu.VMEM((XSUB, FEAT), jnp.float32),
            pltpu.VMEM((WROWS, PK), jnp.int32),
        ],
    )


def kernel(x, y, logits, t, idx, bx, by, bt, blogits):
    xf = x.reshape(BATCH, FEAT)
    bxf = bx.reshape(MEM, FEAT)

    logits_bits = jax.lax.bitcast_convert_type(logits, jnp.int32)
    t_col = jnp.full((BATCH, 1), t, dtype=jnp.int32)
    pad_in = jnp.zeros((BATCH, PK - NCLS - 2), jnp.int32)
    pk_in = jnp.concatenate([logits_bits, y[:, None], t_col, pad_in], axis=1)

    blogits_bits = jax.lax.bitcast_convert_type(blogits, jnp.int32)
    pad_buf = jnp.zeros((MEM, PK - NCLS - 2), jnp.int32)
    pk_buf = jnp.concatenate(
        [blogits_bits, by[:, None], bt[:, None], pad_buf], axis=1)

    kmap = pl.pallas_call(
        _kmap_body,
        grid=(BATCH // KCHUNK,),
        in_specs=[
            pl.BlockSpec((KCHUNK, 1), lambda i: (i, 0)),
            pl.BlockSpec((1, BATCH), lambda i: (0, 0)),
        ],
        out_specs=pl.BlockSpec((KCHUNK, 1), lambda i: (i, 0)),
        out_shape=jax.ShapeDtypeStruct((BATCH, 1), jnp.int32),
    )(idx[:, None], idx[None, :])

    cbx, cpk = _make_copy()(bxf, pk_buf)
    obx_ref = jax.new_ref(cbx)
    opk_ref = jax.new_ref(cpk)
    _make_sc_scatter()(idx[None, :], kmap.reshape(1, BATCH), xf, pk_in,
                       obx_ref, opk_ref)
    obx = obx_ref[...]
    opk = opk_ref[...]

    bx_new = obx.reshape(MEM, 3, 32, 32)
    blogits_new = jax.lax.bitcast_convert_type(opk[:, :NCLS], jnp.float32)
    by_new = opk[:, NCLS]
    bt_new = opk[:, NCLS + 1]
    return (bx_new, by_new, bt_new, blogits_new)


# speedup vs baseline: 1.3085x; 1.2638x over previous
import jax
import jax.numpy as jnp
from jax import lax
from jax.experimental import pallas as pl
from jax.experimental.pallas import tpu as pltpu
from jax.experimental.pallas import tpu_sc as plsc

MEM = 20000
IMG = (3, 32, 32)
FEAT = 3 * 32 * 32
NCLS = 100
PK = 128
BATCH = 4096
COPY_ROWS = 512
KCHUNK = 512
NSUB = 32
WROWS = BATCH // NSUB
XSUB = 32

HALF = MEM // 2
CPCH = 200
PKCH = 1000

_vector_mesh = plsc.VectorSubcoreMesh(
    core_axis_name="core", subcore_axis_name="subcore")


def _make_scalar_mesh():
    return plsc.ScalarSubcoreMesh(axis_name="score", num_cores=2)


def _kmap_body(idx_col_ref, idx_row_ref, out_ref):
    own = idx_col_ref[...]
    allv = idx_row_ref[...]
    iota = lax.broadcasted_iota(jnp.int32, (KCHUNK, BATCH), 1)
    sel = jnp.where(own == allv, iota, -1)
    out_ref[...] = jnp.max(sel, axis=1, keepdims=True)


def _pipe_copy(src, dst, bufs, sems, nch, rows, base):
    def rd(k):
        sl = pl.ds(base + k * rows, rows)
        return pltpu.make_async_copy(src.at[sl], bufs[k % 2], sems[k % 2])

    def wr(k):
        sl = pl.ds(base + k * rows, rows)
        return pltpu.make_async_copy(bufs[k % 2], dst.at[sl], sems[k % 2])

    rd(0).start()
    for k in range(nch):
        rd(k).wait()
        wr(k).start()
        if k + 1 < nch:
            if k >= 1:
                wr(k - 1).wait()
            rd(k + 1).start()
    if nch >= 2:
        wr(nch - 2).wait()
    wr(nch - 1).wait()


def _copy_dma_body(bx_hbm, pkb_hbm, obx_hbm, opk_hbm,
                   xa, xb, pa, pb, sem_a, sem_b, sem_c, sem_d):
    base = lax.axis_index("score") * HALF
    _pipe_copy(bx_hbm, obx_hbm, (xa, xb), (sem_a, sem_b),
               HALF // CPCH, CPCH, base)
    _pipe_copy(pkb_hbm, opk_hbm, (pa, pb), (sem_c, sem_d),
               HALF // PKCH, PKCH, base)


def _make_copy():
    return pl.kernel(
        _copy_dma_body,
        out_type=[
            jax.ShapeDtypeStruct((MEM,) + IMG, jnp.float32),
            jax.ShapeDtypeStruct((MEM, PK), jnp.int32),
        ],
        mesh=_make_scalar_mesh(),
        scratch_types=[
            pltpu.VMEM_SHARED((CPCH,) + IMG, jnp.float32),
            pltpu.VMEM_SHARED((CPCH,) + IMG, jnp.float32),
            pltpu.VMEM_SHARED((PKCH, PK), jnp.int32),
            pltpu.VMEM_SHARED((PKCH, PK), jnp.int32),
            pltpu.SemaphoreType.DMA,
            pltpu.SemaphoreType.DMA,
            pltpu.SemaphoreType.DMA,
            pltpu.SemaphoreType.DMA,
        ],
    )


def _make_sc_scatter():
    def body(idx_hbm, kmap_hbm, x_hbm, pkin_hbm, obx_ref, opk_ref,
             iw_vmem, kw_vmem, xw_vmem, pkw_vmem):
        core = lax.axis_index("core")
        sub = lax.axis_index("subcore")
        off = (core * 16 + sub) * WROWS

        pltpu.sync_copy(idx_hbm.at[0, pl.ds(off, WROWS)], iw_vmem)
        pltpu.sync_copy(kmap_hbm.at[0, pl.ds(off, WROWS)], kw_vmem)

        pltpu.sync_copy(pkin_hbm.at[kw_vmem], pkw_vmem)
        pltpu.sync_copy(pkw_vmem, opk_ref.at[iw_vmem])

        for k in range(WROWS // XSUB):
            sl = pl.ds(k * XSUB, XSUB)
            pltpu.sync_copy(x_hbm.at[kw_vmem.at[sl]], xw_vmem)
            pltpu.sync_copy(xw_vmem, obx_ref.at[iw_vmem.at[sl]])

    return pl.kernel(
        body,
        out_type=(),
        mesh=_vector_mesh,
        scratch_types=[
            pltpu.VMEM((WROWS,), jnp.int32),
            pltpu.VMEM((WROWS,), jnp.int32),
            pltpu.VMEM((XSUB, FEAT), jnp.float32),
            pltpu.VMEM((WROWS, PK), jnp.int32),
        ],
    )


def kernel(x, y, logits, t, idx, bx, by, bt, blogits):

    logits_bits = jax.lax.bitcast_convert_type(logits, jnp.int32)
    t_col = jnp.full((BATCH, 1), t, dtype=jnp.int32)
    pad_in = jnp.zeros((BATCH, PK - NCLS - 2), jnp.int32)
    pk_in = jnp.concatenate([logits_bits, y[:, None], t_col, pad_in], axis=1)

    blogits_bits = jax.lax.bitcast_convert_type(blogits, jnp.int32)
    pad_buf = jnp.zeros((MEM, PK - NCLS - 2), jnp.int32)
    pk_buf = jnp.concatenate(
        [blogits_bits, by[:, None], bt[:, None], pad_buf], axis=1)

    kmap = pl.pallas_call(
            _kmap_body,
            grid=(BATCH // KCHUNK,),
            in_specs=[
                pl.BlockSpec((KCHUNK, 1), lambda i: (i, 0)),
                pl.BlockSpec((1, BATCH), lambda i: (0, 0)),
            ],
            out_specs=pl.BlockSpec((KCHUNK, 1), lambda i: (i, 0)),
            out_shape=jax.ShapeDtypeStruct((BATCH, 1), jnp.int32),
        )(idx[:, None], idx[None, :])

    obx_ref = jax.new_ref(bx.reshape(MEM, FEAT))
    opk_ref = jax.new_ref(pk_buf)
    _make_sc_scatter()(idx[None, :], kmap.reshape(1, BATCH),
                       x.reshape(BATCH, FEAT), pk_in, obx_ref, opk_ref)
    bx_new = jax.freeze(obx_ref).reshape((MEM,) + IMG)
    opk = jax.freeze(opk_ref)

    blogits_new = jax.lax.bitcast_convert_type(opk[:, :NCLS], jnp.float32)
    by_new = opk[:, NCLS]
    bt_new = opk[:, NCLS + 1]
    return (bx_new, by_new, bt_new, blogits_new)
